# baseline (speedup 1.0000x reference)
import jax
import jax.numpy as jnp
from jax.experimental import pallas as pl
from jax.experimental.pallas import tpu as pltpu

K = 64
RB = 128
NC = 10
NF = 5
NE = 3
_NEG = -3.0e38


def _count_ge_bf16(sb, midb):
    n = sb.shape[1]
    parts = [(sb[:, c:c + 128] >= midb).astype(jnp.bfloat16)
             for c in range(0, n, 128)]
    while len(parts) > 1:
        parts = [parts[i] + parts[i + 1] for i in range(0, len(parts), 2)]
    return jnp.sum(parts[0].astype(jnp.float32), axis=1, keepdims=True)


def _body(x_ref, wt_ref, z_ref, s_ref, sb_ref):
    x = x_ref[...]
    xn = x * jax.lax.rsqrt(jnp.maximum(jnp.sum(x * x, axis=1, keepdims=True),
                                       1e-24))
    s = jnp.dot(xn, wt_ref[...], preferred_element_type=jnp.float32)
    s_ref[...] = s
    sb_ref[...] = s.astype(jnp.bfloat16)

    hi = jnp.max(s, axis=1, keepdims=True) + 1e-6
    lo = jnp.zeros_like(hi)

    def cstep(_, carry):
        lo, hi = carry
        midb = ((lo + hi) * 0.5).astype(jnp.bfloat16)
        midf = midb.astype(jnp.float32)
        ge = _count_ge_bf16(sb_ref[...], midb) >= K
        return jnp.where(ge, midf, lo), jnp.where(ge, hi, midf)

    lo, hi = jax.lax.fori_loop(0, NC, cstep, (lo, hi))

    lo = lo - jnp.abs(lo) * 0.0078125 - 1e-9

    def fstep(_, carry):
        lo, hi = carry
        mid = (lo + hi) * 0.5
        ge = jnp.sum((s_ref[...] >= mid).astype(jnp.float32), axis=1,
                     keepdims=True) >= K
        return jnp.where(ge, mid, lo), jnp.where(ge, hi, mid)

    lo, hi = jax.lax.fori_loop(0, NF, fstep, (lo, hi))

    s = s_ref[...]
    c_hi = jnp.sum((s >= hi).astype(jnp.float32), axis=1, keepdims=True)
    j = K - c_hi

    cur = hi
    t = jnp.full_like(hi, _NEG)
    for e in range(1, NE + 1):
        m = jnp.max(jnp.where(s < cur, s, _NEG), axis=1, keepdims=True)
        t = jnp.where(j == e, m, t)
        cur = m
    t = jnp.where(j > NE, cur, t)

    z_ref[...] = jnp.where(s >= t, jnp.maximum(s, 0.0), 0.0)


def kernel(x, W):
    B, D = x.shape
    N = W.shape[0]
    wt = W.T
    return pl.pallas_call(
        _body,
        grid=(B // RB,),
        in_specs=[
            pl.BlockSpec((RB, D), lambda i: (i, 0)),
            pl.BlockSpec((D, N), lambda i: (0, 0)),
        ],
        out_specs=pl.BlockSpec((RB, N), lambda i: (i, 0)),
        out_shape=jax.ShapeDtypeStruct((B, N), jnp.float32),
        scratch_shapes=[
            pltpu.VMEM((RB, N), jnp.float32),
            pltpu.VMEM((RB, N), jnp.bfloat16),
        ],
    )(x, wt)


# speedup vs baseline: 1.4091x; 1.4091x over previous
import jax
import jax.numpy as jnp
from jax.experimental import pallas as pl
from jax.experimental.pallas import tpu as pltpu

K = 64
RB = 128
NF = 13
NE = 4
_NEG = -3.0e38


def _body(x_ref, wt_ref, z_ref, s_ref):
    x = x_ref[...]
    xn = x * jax.lax.rsqrt(jnp.maximum(jnp.sum(x * x, axis=1, keepdims=True),
                                       1e-24))
    s = jnp.dot(xn, wt_ref[...], preferred_element_type=jnp.float32)
    s_ref[...] = s

    hi = jnp.max(s, axis=1, keepdims=True) + 1e-6
    lo = jnp.zeros_like(hi)
    chi = jnp.zeros_like(hi)

    def fstep(_, carry):
        lo, hi, chi = carry
        mid = (lo + hi) * 0.5
        cnt = jnp.sum((s_ref[...] >= mid).astype(jnp.float32), axis=1,
                      keepdims=True)
        ge = cnt >= K
        return (jnp.where(ge, mid, lo), jnp.where(ge, hi, mid),
                jnp.where(ge, chi, cnt))

    lo, hi, chi = jax.lax.fori_loop(0, NF, fstep, (lo, hi, chi))

    s = s_ref[...]
    j = K - chi

    cur = hi
    t = jnp.full_like(hi, _NEG)
    for e in range(1, NE + 1):
        m = jnp.max(jnp.where(s < cur, s, _NEG), axis=1, keepdims=True)
        t = jnp.where(j == e, m, t)
        cur = m
    t = jnp.where(j > NE, cur, t)

    z_ref[...] = jnp.where(s >= t, jnp.maximum(s, 0.0), 0.0)


def kernel(x, W):
    B, D = x.shape
    N = W.shape[0]
    wt = W.T
    return pl.pallas_call(
        _body,
        grid=(B // RB,),
        in_specs=[
            pl.BlockSpec((RB, D), lambda i: (i, 0)),
            pl.BlockSpec((D, N), lambda i: (0, 0)),
        ],
        out_specs=pl.BlockSpec((RB, N), lambda i: (i, 0)),
        out_shape=jax.ShapeDtypeStruct((B, N), jnp.float32),
        scratch_shapes=[pltpu.VMEM((RB, N), jnp.float32)],
    )(x, wt)
